# baseline (speedup 1.0000x reference)
import functools

import jax
import jax.numpy as jnp
from jax import lax
from jax.experimental import pallas as pl
from jax.experimental.pallas import tpu as pltpu
from jax.experimental.pallas import tpu_sc as plsc

BATCH = 4096
SEQ = 200
DIM = 64
N = BATCH * SEQ
NW = 32
PER_W = N // NW
GSZ = 256
NGRP = PER_W // GSZ
NSLOT = 2


@functools.partial(
    pl.kernel,
    mesh=plsc.VectorSubcoreMesh(core_axis_name="c", subcore_axis_name="s"),
    out_type=jax.ShapeDtypeStruct((N, 2 * DIM), jnp.float32),
    scratch_types=[
        pltpu.VMEM((PER_W,), jnp.int32),
        pltpu.VMEM((PER_W,), jnp.int32),
        [pltpu.VMEM((GSZ, DIM), jnp.float32) for _ in range(NSLOT)],
        [pltpu.VMEM((GSZ, DIM), jnp.float32) for _ in range(NSLOT)],
        [pltpu.SemaphoreType.DMA for _ in range(NSLOT)],
        [pltpu.SemaphoreType.DMA for _ in range(NSLOT)],
        [pltpu.SemaphoreType.DMA for _ in range(NSLOT)],
        [pltpu.SemaphoreType.DMA for _ in range(NSLOT)],
    ],
    compiler_params=pltpu.CompilerParams(use_tc_tiling_on_sc=False),
)
def _sc_gather(src_hbm, tgt_hbm, enc_hbm, dec_hbm, out_hbm,
               idx_s, idx_t, re, rd, ge, gd, we, wd):
    wid = lax.axis_index("s") * 2 + lax.axis_index("c")
    base = wid * PER_W

    pltpu.sync_copy(src_hbm.at[pl.ds(base, PER_W)], idx_s)
    pltpu.sync_copy(tgt_hbm.at[pl.ds(base, PER_W)], idx_t)

    def fire_gather(g, b):
        pltpu.async_copy(enc_hbm.at[idx_s.at[pl.ds(g * GSZ, GSZ)]], re[b], ge[b])
        pltpu.async_copy(dec_hbm.at[idx_t.at[pl.ds(g * GSZ, GSZ)]], rd[b], gd[b])

    def wait_gather(g, b):
        pltpu.make_async_copy(enc_hbm.at[idx_s.at[pl.ds(g * GSZ, GSZ)]], re[b], ge[b]).wait()
        pltpu.make_async_copy(dec_hbm.at[idx_t.at[pl.ds(g * GSZ, GSZ)]], rd[b], gd[b]).wait()

    def out_e(g):
        return out_hbm.at[pl.ds(base + g * GSZ, GSZ), pl.ds(0, DIM)]

    def out_d(g):
        return out_hbm.at[pl.ds(base + g * GSZ, GSZ), pl.ds(DIM, DIM)]

    def fire_write(g, b):
        pltpu.async_copy(re[b], out_e(g), we[b])
        pltpu.async_copy(rd[b], out_d(g), wd[b])

    def wait_write(g, b):
        pltpu.make_async_copy(re[b], out_e(g), we[b]).wait()
        pltpu.make_async_copy(rd[b], out_d(g), wd[b]).wait()

    fire_gather(0, 0)

    def body(k, carry):
        g = 2 * k

        @pl.when(k > 0)
        def _():
            wait_write(g - 1, 1)
        fire_gather(g + 1, 1)
        wait_gather(g, 0)
        fire_write(g, 0)

        wait_write(g, 0)

        @pl.when(g + 2 < NGRP)
        def _():
            fire_gather(g + 2, 0)
        wait_gather(g + 1, 1)
        fire_write(g + 1, 1)
        return carry

    lax.fori_loop(0, NGRP // 2, body, 0)
    wait_write(NGRP - 1, 1)


def kernel(src_ids, tgt_ids, enc_table, dec_table):
    out = _sc_gather(src_ids.reshape(N), tgt_ids.reshape(N),
                     enc_table, dec_table)
    return out.reshape(BATCH, SEQ, 2 * DIM)
